# baseline (speedup 1.0000x reference)
import functools

import jax
import jax.numpy as jnp
from jax import lax
from jax.experimental import pallas as pl
from jax.experimental.pallas import tpu as pltpu
from jax.experimental.pallas import tpu_sc as plsc

N = 24
E = 128
LFLAT = N * N


def _sc_body(src_hbm, dst_hbm, tab_hbm, out_hbm, src_v, dst_v, tab_v,
             deg_v, dis_v, l_v):
    cid = lax.axis_index("c")
    sid = lax.axis_index("s")

    @pl.when(jnp.logical_and(cid == 0, sid == 0))
    def _():
        pltpu.sync_copy(src_hbm, src_v)
        pltpu.sync_copy(dst_hbm, dst_v)
        pltpu.sync_copy(tab_hbm, tab_v)
        zeros = jnp.zeros((16,), jnp.float32)
        ones = jnp.ones((16,), jnp.float32)
        deg_v[pl.ds(0, 16)] = zeros
        deg_v[pl.ds(16, 16)] = zeros
        for c in range(LFLAT // 16):
            l_v[pl.ds(c * 16, 16)] = zeros
        for c in range(E // 16):
            s = src_v[pl.ds(c * 16, 16)]
            plsc.addupdate_scatter(deg_v, [s], ones)
        iota = lax.iota(jnp.int32, 16)
        for c in range(2):
            d = deg_v[pl.ds(c * 16, 16)].astype(jnp.int32)
            dis_v[pl.ds(c * 16, 16)] = plsc.load_gather(tab_v, [d])
        for c in range(E // 16):
            s = src_v[pl.ds(c * 16, 16)]
            t = dst_v[pl.ds(c * 16, 16)]
            w = -(plsc.load_gather(dis_v, [s]) * plsc.load_gather(dis_v, [t]))
            plsc.addupdate_scatter(l_v, [t * N + s], w)
        for c in range(2):
            n = iota + c * 16
            d = deg_v[pl.ds(c * 16, 16)]
            dv = jnp.where(d > 0.0, 0.0, -1.0)
            plsc.addupdate_scatter(l_v, [n * (N + 1)], dv, mask=n < N)
        pltpu.sync_copy(l_v, out_hbm)


_sc_build_lhat = functools.partial(
    pl.kernel,
    out_type=jax.ShapeDtypeStruct((LFLAT,), jnp.float32),
    mesh=plsc.VectorSubcoreMesh(core_axis_name="c", subcore_axis_name="s"),
    compiler_params=pltpu.CompilerParams(needs_layout_passes=False),
    scratch_types=[
        pltpu.VMEM((E,), jnp.int32),
        pltpu.VMEM((E,), jnp.int32),
        pltpu.VMEM((136,), jnp.float32),
        pltpu.VMEM((32,), jnp.float32),
        pltpu.VMEM((32,), jnp.float32),
        pltpu.VMEM((LFLAT,), jnp.float32),
    ],
)(_sc_body)


def _elu(v):
    return jnp.where(v > 0, v, jnp.exp(v) - 1.0)


def _mm(a, b):
    return jnp.dot(a.astype(jnp.bfloat16), b.astype(jnp.bfloat16),
                   preferred_element_type=jnp.float32)


def _bf(v):
    return v.astype(jnp.bfloat16).astype(jnp.float32)


def _dense_body(L_ref, x_ref, W1_ref, b1_ref, W2_ref, b2_ref,
                W3_ref, b3_ref, fw_ref, fb_ref, f2w_ref, f2b_ref, o_ref):
    L = L_ref[...]

    def lap(v):
        return jnp.sum(L[:, :, None] * v[None, :, :], axis=1)

    def cheb(h, W_ref, b_ref, K):
        out = _mm(h, W_ref[0])
        Tx0 = h
        Tx1 = lap(h)
        out = out + _mm(Tx1, W_ref[1])
        for k in range(2, K):
            Tx2 = 2.0 * lap(Tx1) - Tx0
            out = out + _mm(Tx2, W_ref[k])
            Tx0, Tx1 = Tx1, Tx2
        return out + b_ref[...][None, :]

    h = _elu(cheb(x_ref[...], W1_ref, b1_ref, 3))
    h = _elu(cheb(h, W2_ref, b2_ref, 3))
    h = _elu(cheb(h, W3_ref, b3_ref, 5))

    prod = _bf(h)[:, :, None] * _bf(fw_ref[...])
    z = jnp.sum(jnp.sum(prod, axis=0), axis=0)
    z = z + fb_ref[...]
    z2 = jnp.sum(_bf(z)[:, None] * _bf(f2w_ref[...]), axis=0)
    z2 = (z2 + f2b_ref[...]).reshape(1, 2)
    m = jnp.max(z2, axis=1, keepdims=True)
    s = z2 - m
    lse = jnp.log(jnp.sum(jnp.exp(s), axis=1, keepdims=True))
    o_ref[...] = s - lse


def kernel(x, edge_index, W1, b1, W2, b2, W3, b3, fc1_w, fc1_b, fc2_w, fc2_b):
    ks = lax.iota(jnp.float32, 136)
    tab = jnp.where(ks > 0, 1.0 / jnp.sqrt(jnp.where(ks > 0, ks, 1.0)), 0.0)
    l_flat = _sc_build_lhat(edge_index[0], edge_index[1], tab)
    L = l_flat.reshape(N, N)
    fw = fc1_w.reshape(N, 32, 128)
    return pl.pallas_call(
        _dense_body,
        out_shape=jax.ShapeDtypeStruct((1, 2), jnp.float32),
    )(L, x, W1, b1, W2, b2, W3, b3, fw, fc1_b, fc2_w, fc2_b)


# speedup vs baseline: 1.0588x; 1.0588x over previous
import functools

import jax
import jax.numpy as jnp
from jax import lax
from jax.experimental import pallas as pl
from jax.experimental.pallas import tpu as pltpu
from jax.experimental.pallas import tpu_sc as plsc

N = 24
E = 128
LFLAT = N * N


def _sc_body(src_hbm, dst_hbm, tab_hbm, out_hbm, src_v, dst_v, tab_v,
             deg_v, dis_v, l_v):
    cid = lax.axis_index("c")
    sid = lax.axis_index("s")

    @pl.when(jnp.logical_and(cid == 0, sid == 0))
    def _():
        pltpu.sync_copy(src_hbm, src_v)
        pltpu.sync_copy(dst_hbm, dst_v)
        pltpu.sync_copy(tab_hbm, tab_v)
        zeros = jnp.zeros((16,), jnp.float32)
        ones = jnp.ones((16,), jnp.float32)
        deg_v[pl.ds(0, 16)] = zeros
        deg_v[pl.ds(16, 16)] = zeros
        for c in range(LFLAT // 16):
            l_v[pl.ds(c * 16, 16)] = zeros
        for c in range(E // 16):
            s = src_v[pl.ds(c * 16, 16)]
            plsc.addupdate_scatter(deg_v, [s], ones)
        iota = lax.iota(jnp.int32, 16)
        for c in range(2):
            d = deg_v[pl.ds(c * 16, 16)].astype(jnp.int32)
            dis_v[pl.ds(c * 16, 16)] = plsc.load_gather(tab_v, [d])
        for c in range(E // 16):
            s = src_v[pl.ds(c * 16, 16)]
            t = dst_v[pl.ds(c * 16, 16)]
            w = -(plsc.load_gather(dis_v, [s]) * plsc.load_gather(dis_v, [t]))
            plsc.addupdate_scatter(l_v, [t * N + s], w)
        for c in range(2):
            n = iota + c * 16
            d = deg_v[pl.ds(c * 16, 16)]
            dv = jnp.where(d > 0.0, 0.0, -1.0)
            plsc.addupdate_scatter(l_v, [n * (N + 1)], dv, mask=n < N)
        pltpu.sync_copy(l_v, out_hbm)


_sc_build_lhat = functools.partial(
    pl.kernel,
    out_type=jax.ShapeDtypeStruct((LFLAT,), jnp.float32),
    mesh=plsc.VectorSubcoreMesh(core_axis_name="c", subcore_axis_name="s",
                                num_cores=1),
    compiler_params=pltpu.CompilerParams(needs_layout_passes=False),
    scratch_types=[
        pltpu.VMEM((E,), jnp.int32),
        pltpu.VMEM((E,), jnp.int32),
        pltpu.VMEM((136,), jnp.float32),
        pltpu.VMEM((32,), jnp.float32),
        pltpu.VMEM((32,), jnp.float32),
        pltpu.VMEM((LFLAT,), jnp.float32),
    ],
)(_sc_body)


def _elu(v):
    return jnp.where(v > 0, v, jnp.exp(v) - 1.0)


def _mm(a, b):
    return jnp.dot(a.astype(jnp.bfloat16), b.astype(jnp.bfloat16),
                   preferred_element_type=jnp.float32)


def _bf(v):
    return v.astype(jnp.bfloat16).astype(jnp.float32)


def _dense_body(L_ref, x_ref, W1_ref, b1_ref, W2_ref, b2_ref,
                W3_ref, b3_ref, fw_ref, fb_ref, f2w_ref, f2b_ref, o_ref):
    L = L_ref[...]

    def lap(v):
        return jnp.sum(L[:, :, None] * v[None, :, :], axis=1)

    def cheb(h, W_ref, b_ref, K):
        out = _mm(h, W_ref[0])
        Tx0 = h
        Tx1 = lap(h)
        out = out + _mm(Tx1, W_ref[1])
        for k in range(2, K):
            Tx2 = 2.0 * lap(Tx1) - Tx0
            out = out + _mm(Tx2, W_ref[k])
            Tx0, Tx1 = Tx1, Tx2
        return out + b_ref[...][None, :]

    h = _elu(cheb(x_ref[...], W1_ref, b1_ref, 3))
    h = _elu(cheb(h, W2_ref, b2_ref, 3))
    h = _elu(cheb(h, W3_ref, b3_ref, 5))

    prod = _bf(h)[:, :, None] * _bf(fw_ref[...])
    z = jnp.sum(jnp.sum(prod, axis=0), axis=0)
    z = z + fb_ref[...]
    z2 = jnp.sum(_bf(z)[:, None] * _bf(f2w_ref[...]), axis=0)
    z2 = (z2 + f2b_ref[...]).reshape(1, 2)
    m = jnp.max(z2, axis=1, keepdims=True)
    s = z2 - m
    lse = jnp.log(jnp.sum(jnp.exp(s), axis=1, keepdims=True))
    o_ref[...] = s - lse


def kernel(x, edge_index, W1, b1, W2, b2, W3, b3, fc1_w, fc1_b, fc2_w, fc2_b):
    ks = lax.iota(jnp.float32, 136)
    tab = jnp.where(ks > 0, 1.0 / jnp.sqrt(jnp.where(ks > 0, ks, 1.0)), 0.0)
    l_flat = _sc_build_lhat(edge_index[0], edge_index[1], tab)
    L = l_flat.reshape(N, N)
    fw = fc1_w.reshape(N, 32, 128)
    return pl.pallas_call(
        _dense_body,
        out_shape=jax.ShapeDtypeStruct((1, 2), jnp.float32),
    )(L, x, W1, b1, W2, b2, W3, b3, fw, fc1_b, fc2_w, fc2_b)


# speedup vs baseline: 1.1012x; 1.0400x over previous
import functools

import jax
import jax.numpy as jnp
from jax import lax
from jax.experimental import pallas as pl
from jax.experimental.pallas import tpu as pltpu
from jax.experimental.pallas import tpu_sc as plsc

N = 24
E = 128
LFLAT = N * N


def _sc_body(buf_hbm, out_hbm, buf_v, deg_v, dis_v, l_v):
    cid = lax.axis_index("c")
    sid = lax.axis_index("s")

    @pl.when(jnp.logical_and(cid == 0, sid == 0))
    def _():
        pltpu.sync_copy(buf_hbm, buf_v)
        zeros = jnp.zeros((16,), jnp.float32)
        ones = jnp.ones((16,), jnp.float32)
        deg_v[pl.ds(0, 16)] = zeros
        deg_v[pl.ds(16, 16)] = zeros
        for c in range(LFLAT // 16):
            l_v[pl.ds(c * 16, 16)] = zeros
        for c in range(E // 16):
            s = buf_v[pl.ds(c * 16, 16)]
            plsc.addupdate_scatter(deg_v, [s], ones)
        iota = lax.iota(jnp.int32, 16)
        for c in range(2):
            d = deg_v[pl.ds(c * 16, 16)].astype(jnp.int32)
            g = plsc.load_gather(buf_v, [d + 2 * E])
            dis_v[pl.ds(c * 16, 16)] = plsc.bitcast(g, jnp.float32)
        for c in range(E // 16):
            s = buf_v[pl.ds(c * 16, 16)]
            t = buf_v[pl.ds(E + c * 16, 16)]
            w = -(plsc.load_gather(dis_v, [s]) * plsc.load_gather(dis_v, [t]))
            plsc.addupdate_scatter(l_v, [t * N + s], w)
        for c in range(2):
            n = iota + c * 16
            d = deg_v[pl.ds(c * 16, 16)]
            dv = jnp.where(d > 0.0, 0.0, -1.0)
            plsc.addupdate_scatter(l_v, [n * (N + 1)], dv, mask=n < N)
        pltpu.sync_copy(l_v, out_hbm)


_sc_build_lhat = functools.partial(
    pl.kernel,
    out_type=jax.ShapeDtypeStruct((LFLAT,), jnp.float32),
    mesh=plsc.VectorSubcoreMesh(core_axis_name="c", subcore_axis_name="s",
                                num_cores=1),
    compiler_params=pltpu.CompilerParams(needs_layout_passes=False),
    scratch_types=[
        pltpu.VMEM((2 * E + 136,), jnp.int32),
        pltpu.VMEM((32,), jnp.float32),
        pltpu.VMEM((32,), jnp.float32),
        pltpu.VMEM((LFLAT,), jnp.float32),
    ],
)(_sc_body)


def _elu(v):
    return jnp.where(v > 0, v, jnp.exp(v) - 1.0)


def _mm(a, b):
    return jnp.dot(a.astype(jnp.bfloat16), b.astype(jnp.bfloat16),
                   preferred_element_type=jnp.float32)


def _bf(v):
    return v.astype(jnp.bfloat16).astype(jnp.float32)


def _dense_body(L_ref, x_ref, W1_ref, b1_ref, W2_ref, b2_ref,
                W3_ref, b3_ref, fw_ref, fb_ref, f2w_ref, f2b_ref, o_ref):
    L = L_ref[...]

    def lap(v):
        return jnp.sum(L[:, :, None] * v[None, :, :], axis=1)

    def cheb(h, W_ref, b_ref, K):
        out = _mm(h, W_ref[0])
        Tx0 = h
        Tx1 = lap(h)
        out = out + _mm(Tx1, W_ref[1])
        for k in range(2, K):
            Tx2 = 2.0 * lap(Tx1) - Tx0
            out = out + _mm(Tx2, W_ref[k])
            Tx0, Tx1 = Tx1, Tx2
        return out + b_ref[...][None, :]

    h = _elu(cheb(x_ref[...], W1_ref, b1_ref, 3))
    h = _elu(cheb(h, W2_ref, b2_ref, 3))
    h = _elu(cheb(h, W3_ref, b3_ref, 5))

    prod = _bf(h)[:, :, None] * _bf(fw_ref[...])
    z = jnp.sum(jnp.sum(prod, axis=0), axis=0)
    z = z + fb_ref[...]
    z2 = jnp.sum(_bf(z)[:, None] * _bf(f2w_ref[...]), axis=0)
    z2 = (z2 + f2b_ref[...]).reshape(1, 2)
    m = jnp.max(z2, axis=1, keepdims=True)
    s = z2 - m
    lse = jnp.log(jnp.sum(jnp.exp(s), axis=1, keepdims=True))
    o_ref[...] = s - lse


def kernel(x, edge_index, W1, b1, W2, b2, W3, b3, fc1_w, fc1_b, fc2_w, fc2_b):
    ks = lax.iota(jnp.float32, 136)
    tab = jnp.where(ks > 0, 1.0 / jnp.sqrt(jnp.where(ks > 0, ks, 1.0)), 0.0)
    buf = jnp.concatenate([edge_index[0], edge_index[1],
                           lax.bitcast_convert_type(tab, jnp.int32)])
    l_flat = _sc_build_lhat(buf)
    L = l_flat.reshape(N, N)
    fw = fc1_w.reshape(N, 32, 128)
    return pl.pallas_call(
        _dense_body,
        out_shape=jax.ShapeDtypeStruct((1, 2), jnp.float32),
    )(L, x, W1, b1, W2, b2, W3, b3, fw, fc1_b, fc2_w, fc2_b)
